# baseline (speedup 1.0000x reference)
import functools

import jax
import jax.numpy as jnp
from jax import lax
from jax.experimental import pallas as pl
from jax.experimental.pallas import tpu as pltpu
from jax.experimental.pallas import tpu_sc as plsc

SOM0 = 32
SOM1 = 32
CODE_DIM = 32
NCODES = SOM0 * SOM1
BLOCK_B = 512

NC = 2
NS = 16
NW = NC * NS
B_TOTAL = 2048
BPW = B_TOTAL // NW

_HIGH = jax.lax.Precision.HIGHEST


def _tc_body(z_ref, embt_ref, emb_ref,
             zdist_ref, k_ref, zq_ref, zqst_ref, csum_ref, en_ref):
    z = z_ref[...]

    @pl.when(pl.program_id(0) == 0)
    def _compute_en():
        embt = embt_ref[...]
        en_ref[...] = jnp.sum(embt * embt, axis=0, keepdims=True)

    en = en_ref[...]
    dot = jnp.dot(z, embt_ref[...], preferred_element_type=jnp.float32,
                  precision=_HIGH)
    s = en - 2.0 * dot
    zn = jnp.sum(z * z, axis=1, keepdims=True)
    zdist_ref[...] = zn + s

    idx = jax.lax.broadcasted_iota(jnp.int32, s.shape, 1)
    m1 = jnp.min(s, axis=1, keepdims=True)
    j1 = jnp.min(jnp.where(s == m1, idx, NCODES), axis=1, keepdims=True)
    s2 = jnp.where(idx == j1, jnp.float32(3e38), s)
    m2 = jnp.min(s2, axis=1, keepdims=True)
    j2 = jnp.min(jnp.where(s2 == m2, idx, NCODES), axis=1, keepdims=True)

    emb = emb_ref[...]
    oh1 = (idx == j1).astype(jnp.float32)
    oh2 = (idx == j2).astype(jnp.float32)
    e1 = jnp.dot(oh1, emb, preferred_element_type=jnp.float32,
                 precision=_HIGH)
    e2 = jnp.dot(oh2, emb, preferred_element_type=jnp.float32,
                 precision=_HIGH)
    d1 = jnp.sum((z - e1) ** 2, axis=1, keepdims=True)
    d2 = jnp.sum((z - e2) ** 2, axis=1, keepdims=True)
    take2 = (d2 < d1) | ((d2 == d1) & (j2 < j1))
    k = jnp.where(take2, j2, j1)
    k_ref[...] = k
    zq = jnp.where(take2, e2, e1)
    zq_ref[...] = zq
    zqst_ref[...] = z + (zq - z)

    part = jnp.sum((zq - z) ** 2)

    @pl.when(pl.program_id(0) == 0)
    def _init():
        csum_ref[...] = jnp.zeros((1, 1), jnp.float32)

    csum_ref[...] += part.reshape(1, 1)


def _tc_stage(z_e, embt, emb):
    b = z_e.shape[0]
    grid = (b // BLOCK_B,)
    out_shapes = (
        jax.ShapeDtypeStruct((b, NCODES), jnp.float32),
        jax.ShapeDtypeStruct((b, 1), jnp.int32),
        jax.ShapeDtypeStruct((b, CODE_DIM), jnp.float32),
        jax.ShapeDtypeStruct((b, CODE_DIM), jnp.float32),
        jax.ShapeDtypeStruct((1, 1), jnp.float32),
    )
    row_spec = pl.BlockSpec((BLOCK_B, CODE_DIM), lambda i: (i, 0))
    out_specs = (
        pl.BlockSpec((BLOCK_B, NCODES), lambda i: (i, 0)),
        pl.BlockSpec((BLOCK_B, 1), lambda i: (i, 0)),
        row_spec, row_spec,
        pl.BlockSpec((1, 1), lambda i: (0, 0)),
    )
    in_specs = (
        row_spec,
        pl.BlockSpec((CODE_DIM, NCODES), lambda i: (0, 0)),
        pl.BlockSpec((NCODES, CODE_DIM), lambda i: (0, 0)),
    )
    return pl.pallas_call(
        _tc_body,
        grid=grid,
        in_specs=in_specs,
        out_specs=out_specs,
        out_shape=out_shapes,
        scratch_shapes=[pltpu.VMEM((1, NCODES), jnp.float32)],
    )(z_e, embt, emb)


_sc_mesh = plsc.VectorSubcoreMesh(
    core_axis_name="c", subcore_axis_name="s", num_cores=NC, num_subcores=NS)


@functools.partial(
    pl.kernel,
    out_type=(
        jax.ShapeDtypeStruct((B_TOTAL, CODE_DIM), jnp.float32),
        jax.ShapeDtypeStruct((B_TOTAL, CODE_DIM), jnp.float32),
        jax.ShapeDtypeStruct((B_TOTAL, CODE_DIM), jnp.float32),
        jax.ShapeDtypeStruct((B_TOTAL, CODE_DIM), jnp.float32),
        jax.ShapeDtypeStruct((NW, 16), jnp.float32),
    ),
    mesh=_sc_mesh,
    compiler_params=pltpu.CompilerParams(use_tc_tiling_on_sc=False),
    scratch_types=[
        pltpu.VMEM((BPW,), jnp.int32),
        pltpu.VMEM((BPW,), jnp.int32),
        pltpu.VMEM((BPW,), jnp.int32),
        pltpu.VMEM((BPW,), jnp.int32),
        pltpu.VMEM((BPW,), jnp.int32),
        pltpu.VMEM((BPW, CODE_DIM), jnp.float32),
        pltpu.VMEM((BPW, CODE_DIM), jnp.float32),
        pltpu.VMEM((BPW, CODE_DIM), jnp.float32),
        pltpu.VMEM((BPW, CODE_DIM), jnp.float32),
        pltpu.VMEM((BPW, CODE_DIM), jnp.float32),
        pltpu.VMEM((16,), jnp.float32),
        pltpu.SemaphoreType.DMA,
    ],
)
def _sc_gather(emb_hbm, k_hbm, z_hbm,
               up_hbm, dn_hbm, rt_hbm, lf_hbm, part_hbm,
               kv, iu, idn, irt, ilf, ru, rdn, rrt, rlf, zv, accv, sem):
    wid = lax.axis_index("s") * NC + lax.axis_index("c")
    base = wid * BPW
    pltpu.sync_copy(k_hbm.at[pl.ds(base, BPW)], kv)
    for ch in range(BPW // 16):
        sl = pl.ds(ch * 16, 16)
        kk = kv[sl]
        k1 = kk >> 5
        k2 = kk & 31
        iu[sl] = jnp.where(k1 < SOM0 - 1, kk + SOM1, NCODES)
        idn[sl] = jnp.where(k1 > 0, kk - SOM1, NCODES)
        irt[sl] = jnp.where(k2 < SOM1 - 1, kk + 1, NCODES)
        ilf[sl] = jnp.where(k2 > 0, kk - 1, NCODES)
    cps = [pltpu.async_copy(emb_hbm.at[iu], ru, sem),
           pltpu.async_copy(emb_hbm.at[idn], rdn, sem),
           pltpu.async_copy(emb_hbm.at[irt], rrt, sem),
           pltpu.async_copy(emb_hbm.at[ilf], rlf, sem)]
    pltpu.sync_copy(z_hbm.at[pl.ds(base, BPW)], zv)
    for cp in cps:
        cp.wait()
    pltpu.sync_copy(ru, up_hbm.at[pl.ds(base, BPW)])
    pltpu.sync_copy(rdn, dn_hbm.at[pl.ds(base, BPW)])
    pltpu.sync_copy(rrt, rt_hbm.at[pl.ds(base, BPW)])
    pltpu.sync_copy(rlf, lf_hbm.at[pl.ds(base, BPW)])

    def body(r, acc):
        z0 = zv[r, pl.ds(0, 16)]
        z1 = zv[r, pl.ds(16, 16)]
        for rows in (ru, rdn, rrt, rlf):
            d0 = z0 - rows[r, pl.ds(0, 16)]
            d1 = z1 - rows[r, pl.ds(16, 16)]
            acc = acc + d0 * d0 + d1 * d1
        return acc

    acc = lax.fori_loop(0, BPW, body, jnp.zeros((16,), jnp.float32))
    accv[...] = acc
    pltpu.sync_copy(accv, part_hbm.at[wid])


@functools.partial(jax.jit, static_argnames=())
def kernel(x, embeddings):
    n, c, t = x.shape
    b = n * t
    z_e = jnp.transpose(x, (0, 2, 1)).reshape(b, c)
    emb = embeddings.reshape(NCODES, CODE_DIM)
    embt = emb.T
    emb_aug = jnp.concatenate(
        [emb, jnp.zeros((1, CODE_DIM), jnp.float32)], axis=0)

    z_dist, k2d, z_q, z_q_st, csum = _tc_stage(z_e, embt, emb)
    k = k2d.reshape(b)

    up, down, right, left, part = _sc_gather(emb_aug, k, z_e)

    z_q_neighbors = jnp.stack([z_q, up, down, right, left], axis=1)
    commit_l = 2.0 * (csum[0, 0] / jnp.float32(b * c))
    som_l = (csum[0, 0] + jnp.sum(part)) / jnp.float32(b * 5 * c)
    z_q_out = jnp.transpose(z_q_st.reshape(n, t, c), (0, 2, 1))
    return (z_q_out, commit_l, som_l, z_q_neighbors, z_dist, k)


# speedup vs baseline: 1.0044x; 1.0044x over previous
import functools

import jax
import jax.numpy as jnp
from jax import lax
from jax.experimental import pallas as pl
from jax.experimental.pallas import tpu as pltpu
from jax.experimental.pallas import tpu_sc as plsc

SOM0 = 32
SOM1 = 32
CODE_DIM = 32
NCODES = SOM0 * SOM1
BLOCK_B = 1024

NC = 2
NS = 16
NW = NC * NS
B_TOTAL = 2048
BPW = B_TOTAL // NW

_HIGH = jax.lax.Precision.HIGHEST


def _tc_body(z_ref, embt_ref, emb_ref,
             zdist_ref, k_ref, zq_ref, zqst_ref, csum_ref, en_ref):
    z = z_ref[...]

    @pl.when(pl.program_id(0) == 0)
    def _compute_en():
        embt = embt_ref[...]
        en_ref[...] = jnp.sum(embt * embt, axis=0, keepdims=True)

    en = en_ref[...]
    dot = jnp.dot(z, embt_ref[...], preferred_element_type=jnp.float32,
                  precision=_HIGH)
    s = en - 2.0 * dot
    zn = jnp.sum(z * z, axis=1, keepdims=True)
    zdist_ref[...] = zn + s

    idx = jax.lax.broadcasted_iota(jnp.int32, s.shape, 1)
    m1 = jnp.min(s, axis=1, keepdims=True)
    j1 = jnp.min(jnp.where(s == m1, idx, NCODES), axis=1, keepdims=True)
    s2 = jnp.where(idx == j1, jnp.float32(3e38), s)
    m2 = jnp.min(s2, axis=1, keepdims=True)
    j2 = jnp.min(jnp.where(s2 == m2, idx, NCODES), axis=1, keepdims=True)

    emb = emb_ref[...]
    oh1 = (idx == j1).astype(jnp.float32)
    oh2 = (idx == j2).astype(jnp.float32)
    e1 = jnp.dot(oh1, emb, preferred_element_type=jnp.float32,
                 precision=_HIGH)
    e2 = jnp.dot(oh2, emb, preferred_element_type=jnp.float32,
                 precision=_HIGH)
    d1 = jnp.sum((z - e1) ** 2, axis=1, keepdims=True)
    d2 = jnp.sum((z - e2) ** 2, axis=1, keepdims=True)
    take2 = (d2 < d1) | ((d2 == d1) & (j2 < j1))
    k = jnp.where(take2, j2, j1)
    k_ref[...] = k
    zq = jnp.where(take2, e2, e1)
    zq_ref[...] = zq
    zqst_ref[...] = z + (zq - z)

    part = jnp.sum((zq - z) ** 2)

    @pl.when(pl.program_id(0) == 0)
    def _init():
        csum_ref[...] = jnp.zeros((1, 1), jnp.float32)

    csum_ref[...] += part.reshape(1, 1)


def _tc_stage(z_e, embt, emb):
    b = z_e.shape[0]
    grid = (b // BLOCK_B,)
    out_shapes = (
        jax.ShapeDtypeStruct((b, NCODES), jnp.float32),
        jax.ShapeDtypeStruct((b, 1), jnp.int32),
        jax.ShapeDtypeStruct((b, CODE_DIM), jnp.float32),
        jax.ShapeDtypeStruct((b, CODE_DIM), jnp.float32),
        jax.ShapeDtypeStruct((1, 1), jnp.float32),
    )
    row_spec = pl.BlockSpec((BLOCK_B, CODE_DIM), lambda i: (i, 0))
    out_specs = (
        pl.BlockSpec((BLOCK_B, NCODES), lambda i: (i, 0)),
        pl.BlockSpec((BLOCK_B, 1), lambda i: (i, 0)),
        row_spec, row_spec,
        pl.BlockSpec((1, 1), lambda i: (0, 0)),
    )
    in_specs = (
        row_spec,
        pl.BlockSpec((CODE_DIM, NCODES), lambda i: (0, 0)),
        pl.BlockSpec((NCODES, CODE_DIM), lambda i: (0, 0)),
    )
    return pl.pallas_call(
        _tc_body,
        grid=grid,
        in_specs=in_specs,
        out_specs=out_specs,
        out_shape=out_shapes,
        scratch_shapes=[pltpu.VMEM((1, NCODES), jnp.float32)],
    )(z_e, embt, emb)


_sc_mesh = plsc.VectorSubcoreMesh(
    core_axis_name="c", subcore_axis_name="s", num_cores=NC, num_subcores=NS)


@functools.partial(
    pl.kernel,
    out_type=(
        jax.ShapeDtypeStruct((B_TOTAL, CODE_DIM), jnp.float32),
        jax.ShapeDtypeStruct((B_TOTAL, CODE_DIM), jnp.float32),
        jax.ShapeDtypeStruct((B_TOTAL, CODE_DIM), jnp.float32),
        jax.ShapeDtypeStruct((B_TOTAL, CODE_DIM), jnp.float32),
        jax.ShapeDtypeStruct((NW, 16), jnp.float32),
    ),
    mesh=_sc_mesh,
    compiler_params=pltpu.CompilerParams(use_tc_tiling_on_sc=False),
    scratch_types=[
        pltpu.VMEM((BPW,), jnp.int32),
        pltpu.VMEM((BPW,), jnp.int32),
        pltpu.VMEM((BPW,), jnp.int32),
        pltpu.VMEM((BPW,), jnp.int32),
        pltpu.VMEM((BPW,), jnp.int32),
        pltpu.VMEM((BPW, CODE_DIM), jnp.float32),
        pltpu.VMEM((BPW, CODE_DIM), jnp.float32),
        pltpu.VMEM((BPW, CODE_DIM), jnp.float32),
        pltpu.VMEM((BPW, CODE_DIM), jnp.float32),
        pltpu.VMEM((BPW, CODE_DIM), jnp.float32),
        pltpu.VMEM((16,), jnp.float32),
        pltpu.SemaphoreType.DMA,
    ],
)
def _sc_gather(emb_hbm, k_hbm, z_hbm,
               up_hbm, dn_hbm, rt_hbm, lf_hbm, part_hbm,
               kv, iu, idn, irt, ilf, ru, rdn, rrt, rlf, zv, accv, sem):
    wid = lax.axis_index("s") * NC + lax.axis_index("c")
    base = wid * BPW
    pltpu.sync_copy(k_hbm.at[pl.ds(base, BPW)], kv)
    for ch in range(BPW // 16):
        sl = pl.ds(ch * 16, 16)
        kk = kv[sl]
        k1 = kk >> 5
        k2 = kk & 31
        iu[sl] = jnp.where(k1 < SOM0 - 1, kk + SOM1, NCODES)
        idn[sl] = jnp.where(k1 > 0, kk - SOM1, NCODES)
        irt[sl] = jnp.where(k2 < SOM1 - 1, kk + 1, NCODES)
        ilf[sl] = jnp.where(k2 > 0, kk - 1, NCODES)
    cps = [pltpu.async_copy(emb_hbm.at[iu], ru, sem),
           pltpu.async_copy(emb_hbm.at[idn], rdn, sem),
           pltpu.async_copy(emb_hbm.at[irt], rrt, sem),
           pltpu.async_copy(emb_hbm.at[ilf], rlf, sem)]
    pltpu.sync_copy(z_hbm.at[pl.ds(base, BPW)], zv)
    for cp in cps:
        cp.wait()
    pltpu.sync_copy(ru, up_hbm.at[pl.ds(base, BPW)])
    pltpu.sync_copy(rdn, dn_hbm.at[pl.ds(base, BPW)])
    pltpu.sync_copy(rrt, rt_hbm.at[pl.ds(base, BPW)])
    pltpu.sync_copy(rlf, lf_hbm.at[pl.ds(base, BPW)])

    def body(r, acc):
        z0 = zv[r, pl.ds(0, 16)]
        z1 = zv[r, pl.ds(16, 16)]
        for rows in (ru, rdn, rrt, rlf):
            d0 = z0 - rows[r, pl.ds(0, 16)]
            d1 = z1 - rows[r, pl.ds(16, 16)]
            acc = acc + d0 * d0 + d1 * d1
        return acc

    acc = lax.fori_loop(0, BPW, body, jnp.zeros((16,), jnp.float32))
    accv[...] = acc
    pltpu.sync_copy(accv, part_hbm.at[wid])


@functools.partial(jax.jit, static_argnames=())
def kernel(x, embeddings):
    n, c, t = x.shape
    b = n * t
    z_e = jnp.transpose(x, (0, 2, 1)).reshape(b, c)
    emb = embeddings.reshape(NCODES, CODE_DIM)
    embt = emb.T
    emb_aug = jnp.concatenate(
        [emb, jnp.zeros((1, CODE_DIM), jnp.float32)], axis=0)

    z_dist, k2d, z_q, z_q_st, csum = _tc_stage(z_e, embt, emb)
    k = k2d.reshape(b)

    up, down, right, left, part = _sc_gather(emb_aug, k, z_e)

    z_q_neighbors = jnp.stack([z_q, up, down, right, left], axis=1)
    commit_l = 2.0 * (csum[0, 0] / jnp.float32(b * c))
    som_l = (csum[0, 0] + jnp.sum(part)) / jnp.float32(b * 5 * c)
    z_q_out = jnp.transpose(z_q_st.reshape(n, t, c), (0, 2, 1))
    return (z_q_out, commit_l, som_l, z_q_neighbors, z_dist, k)
